# baseline (speedup 1.0000x reference)
import functools

import jax
import jax.numpy as jnp
from jax import lax
from jax.experimental import pallas as pl
from jax.experimental.pallas import tpu as pltpu
from jax.experimental.pallas import tpu_sc as plsc

N = 10000
NP = 10240
PAD = NP - N
E = 320000
NUM_GRAPHS = 64
NC = 2
NS = 16
NW = NC * NS
EPT = E // NW
CHUNK = 80
NCHUNK = EPT // CHUNK
ROWS_PT = NP // NS
DEGW = 16
D = 5
NG = NCHUNK // D


def _mesh():
    return plsc.VectorSubcoreMesh(
        core_axis_name="c", subcore_axis_name="s",
        num_cores=NC, num_subcores=NS)


def _deg_body(dst3d_hbm, degp_hbm, dstb, ones_v, zbuf, deg_sp, sd):
    c = lax.axis_index("c")
    s = lax.axis_index("s")
    wid = c * NS + s

    def fill(i, _):
        zbuf[i] = jnp.zeros((DEGW,), jnp.float32)
        return 0
    lax.fori_loop(0, ROWS_PT, fill, 0)

    def fill1(i, _):
        ones_v[i] = jnp.ones((DEGW,), jnp.float32)
        return 0
    lax.fori_loop(0, CHUNK, fill1, 0)

    pltpu.sync_copy(zbuf, deg_sp.at[pl.ds(s * ROWS_PT, ROWS_PT)])
    pltpu.sync_copy(dst3d_hbm.at[wid], dstb)
    plsc.subcore_barrier()

    def body(j, _):
        pltpu.sync_copy(ones_v, deg_sp.at[dstb.at[j]], add=True)
        return 0
    lax.fori_loop(0, NCHUNK, body, 0)

    plsc.subcore_barrier()
    pltpu.sync_copy(deg_sp.at[pl.ds(s * ROWS_PT, ROWS_PT)],
                    degp_hbm.at[c, pl.ds(s * ROWS_PT, ROWS_PT)])


def _deg(dst3d):
    return pl.kernel(
        _deg_body,
        out_type=jax.ShapeDtypeStruct((NC, NP, DEGW), jnp.float32),
        mesh=_mesh(),
        compiler_params=pltpu.CompilerParams(use_tc_tiling_on_sc=False),
        scratch_types=[
            pltpu.VMEM((NCHUNK, CHUNK), jnp.int32),
            pltpu.VMEM((CHUNK, DEGW), jnp.float32),
            pltpu.VMEM((ROWS_PT, DEGW), jnp.float32),
            pltpu.VMEM_SHARED((NP, DEGW), jnp.float32),
            pltpu.SemaphoreType.DMA,
        ],
    )(dst3d)


def _prop_body(hs_hbm, src3d_hbm, dst3d_hbm, part_hbm,
               srcb, dstb, rows, acc_sp, g0, g1, s0, s1):
    c = lax.axis_index("c")
    s = lax.axis_index("s")
    wid = c * NS + s

    pltpu.sync_copy(hs_hbm.at[pl.ds(s * ROWS_PT, ROWS_PT)],
                    acc_sp.at[pl.ds(s * ROWS_PT, ROWS_PT)])
    pltpu.sync_copy(src3d_hbm.at[wid], srcb)
    pltpu.sync_copy(dst3d_hbm.at[wid], dstb)
    plsc.subcore_barrier()

    pltpu.async_copy(hs_hbm.at[srcb.at[0]], rows.at[0], g0)

    npair = (NCHUNK - 1) // 2

    def body(jj, _):
        j0 = 2 * jj
        j1 = j0 + 1
        pltpu.async_copy(hs_hbm.at[srcb.at[j1]], rows.at[1], g1)
        pltpu.make_async_copy(hs_hbm.at[srcb.at[j0]], rows.at[0], g0).wait()
        pltpu.sync_copy(rows.at[0], acc_sp.at[dstb.at[j0]], add=True)
        pltpu.async_copy(hs_hbm.at[srcb.at[j0 + 2]], rows.at[0], g0)
        pltpu.make_async_copy(hs_hbm.at[srcb.at[j1]], rows.at[1], g1).wait()
        pltpu.sync_copy(rows.at[1], acc_sp.at[dstb.at[j1]], add=True)
        return 0
    lax.fori_loop(0, npair, body, 0)

    last = NCHUNK - 1
    pltpu.make_async_copy(hs_hbm.at[srcb.at[last]], rows.at[0], g0).wait()
    pltpu.sync_copy(rows.at[0], acc_sp.at[dstb.at[last]], add=True)

    plsc.subcore_barrier()
    pltpu.sync_copy(acc_sp.at[pl.ds(s * ROWS_PT, ROWS_PT)],
                    part_hbm.at[c, pl.ds(s * ROWS_PT, ROWS_PT)])


def _prop(hs, src3d, dst3d, w):
    return pl.kernel(
        _prop_body,
        out_type=jax.ShapeDtypeStruct((NC, NP, w), jnp.float32),
        mesh=_mesh(),
        compiler_params=pltpu.CompilerParams(use_tc_tiling_on_sc=False),
        scratch_types=[
            pltpu.VMEM((NCHUNK, CHUNK), jnp.int32),
            pltpu.VMEM((NCHUNK, CHUNK), jnp.int32),
            pltpu.VMEM((2, CHUNK, w), jnp.float32),
            pltpu.VMEM_SHARED((NP, w), jnp.float32),
            pltpu.SemaphoreType.DMA,
            pltpu.SemaphoreType.DMA,
            pltpu.SemaphoreType.DMA,
            pltpu.SemaphoreType.DMA,
        ],
    )(hs, src3d, dst3d)


def _dinv_from(degp):
    deg = degp[0] + degp[1]
    return lax.rsqrt(deg[:, :1] + 1.0)


def _mm_scale_body(x_ref, w_ref, degp_ref, hs_ref):
    dinv = _dinv_from(degp_ref[...])
    xw = jnp.dot(x_ref[...], w_ref[...], preferred_element_type=jnp.float32)
    hs_ref[pl.ds(0, N), :] = xw * dinv[:N]
    hs_ref[pl.ds(N, PAD), :] = jnp.zeros((PAD, 64), jnp.float32)


def _combine_mm_body(a_ref, hs_ref, degp_ref, b1_ref, w2_ref, out_ref):
    dinv = _dinv_from(degp_ref[...])
    a = a_ref[...]
    hs = hs_ref[...]
    p = (a[0] + a[1] - hs) * dinv + b1_ref[...]
    h1 = jnp.maximum(p, 0.0)
    hh = jnp.dot(h1, w2_ref[...], preferred_element_type=jnp.float32)
    out_ref[...] = hh * dinv
    out_ref[pl.ds(N, PAD), :] = jnp.zeros((PAD, 128), jnp.float32)


def _leaky(v, alpha):
    return jnp.where(v > 0, v, alpha * v)


def _head_body(a_ref, hs_ref, degp_ref, b2_ref, batch_ref,
               m0_ref, mb0_ref, m1_ref, mb1_ref, m2_ref, mb2_ref,
               m3_ref, mb3_ref, out_ref):
    dinv = _dinv_from(degp_ref[...])
    a = a_ref[...]
    p2 = (a[0] + a[1] - hs_ref[...]) * dinv + b2_ref[...]
    h2 = jnp.maximum(p2, 0.0)
    gids = lax.broadcasted_iota(jnp.int32, (NUM_GRAPHS, NP), 0)
    onehot = (batch_ref[...] == gids).astype(jnp.float32)
    sums = jnp.dot(onehot, h2, preferred_element_type=jnp.float32,
                   precision=lax.Precision.HIGHEST)
    cnt = jnp.sum(onehot, axis=1, keepdims=True)
    g = sums / jnp.maximum(cnt, 1.0)
    g = _leaky(jnp.dot(g, m0_ref[...], preferred_element_type=jnp.float32)
               + mb0_ref[...], 0.2)
    g = _leaky(jnp.dot(g, m1_ref[...], preferred_element_type=jnp.float32)
               + mb1_ref[...], 0.1)
    g = _leaky(jnp.dot(g, m2_ref[...], preferred_element_type=jnp.float32)
               + mb2_ref[...], 0.1)
    g = jnp.dot(g, m3_ref[...], preferred_element_type=jnp.float32)
    out_ref[...] = jnp.maximum(g + mb3_ref[...], 0.0)


def _tc_call(body, out_shape, *args):
    return pl.pallas_call(
        body,
        out_shape=jax.ShapeDtypeStruct(out_shape, jnp.float32),
    )(*args)


def kernel(x, edge_index, batch, W1, b1, W2, b2,
           M0, mb0, M1, mb1, M2, mb2, M3, mb3):
    src3d = edge_index[0].reshape(NW, NCHUNK, CHUNK)
    dst3d = edge_index[1].reshape(NW, NCHUNK, CHUNK)
    batch_pad = jnp.concatenate(
        [batch, jnp.full((PAD,), -1, jnp.int32)]).reshape(1, NP)

    degp = _deg(dst3d)
    hs1 = _tc_call(_mm_scale_body, (NP, 64),
                   x, W1, degp)
    part1 = _prop(hs1, src3d, dst3d, 64)
    hs2 = _tc_call(_combine_mm_body, (NP, 128),
                   part1, hs1, degp, b1.reshape(1, 64),
                   W2)
    part2 = _prop(hs2, src3d, dst3d, 128)
    out = _tc_call(_head_body, (NUM_GRAPHS, 1),
                   part2, hs2, degp, b2.reshape(1, 128), batch_pad,
                   M0, mb0.reshape(1, 64), M1, mb1.reshape(1, 64),
                   M2, mb2.reshape(1, 64), M3, mb3.reshape(1, 1))
    return out


# speedup vs baseline: 1.1089x; 1.1089x over previous
import functools

import jax
import jax.numpy as jnp
from jax import lax
from jax.experimental import pallas as pl
from jax.experimental.pallas import tpu as pltpu
from jax.experimental.pallas import tpu_sc as plsc

N = 10000
NP = 10240
PAD = NP - N
E = 320000
NUM_GRAPHS = 64
NC = 2
NS = 16
NW = NC * NS
EPT = E // NW
CHUNK = 80
NCHUNK = EPT // CHUNK
ROWS_PT = NP // NS
DEGW = 16
D = 5
NG = NCHUNK // D


def _mesh():
    return plsc.VectorSubcoreMesh(
        core_axis_name="c", subcore_axis_name="s",
        num_cores=NC, num_subcores=NS)


def _deg_body(dst3d_hbm, degp_hbm, dstb, ones_v, zbuf, deg_sp, sd):
    c = lax.axis_index("c")
    s = lax.axis_index("s")
    wid = c * NS + s

    def fill(i, _):
        zbuf[i] = jnp.zeros((DEGW,), jnp.float32)
        return 0
    lax.fori_loop(0, ROWS_PT, fill, 0)

    def fill1(i, _):
        ones_v[i] = jnp.ones((DEGW,), jnp.float32)
        return 0
    lax.fori_loop(0, CHUNK, fill1, 0)

    pltpu.sync_copy(zbuf, deg_sp.at[pl.ds(s * ROWS_PT, ROWS_PT)])
    pltpu.sync_copy(dst3d_hbm.at[wid], dstb)
    plsc.subcore_barrier()

    for d in range(D):
        pltpu.async_copy(ones_v, deg_sp.at[dstb.at[d]], sd, add=True)

    def body(i, _):
        for d in range(D):
            pltpu.async_copy(ones_v, deg_sp.at[dstb.at[(i + 1) * D + d]],
                             sd, add=True)
        for d in range(D):
            pltpu.make_async_copy(ones_v, deg_sp.at[dstb.at[i * D + d]],
                                  sd).wait()
        return 0
    lax.fori_loop(0, NG - 1, body, 0)

    lastd = (NG - 1) * D
    for d in range(D):
        pltpu.make_async_copy(ones_v, deg_sp.at[dstb.at[lastd + d]],
                              sd).wait()

    plsc.subcore_barrier()
    pltpu.sync_copy(deg_sp.at[pl.ds(s * ROWS_PT, ROWS_PT)],
                    degp_hbm.at[c, pl.ds(s * ROWS_PT, ROWS_PT)])


def _deg(dst3d):
    return pl.kernel(
        _deg_body,
        out_type=jax.ShapeDtypeStruct((NC, NP, DEGW), jnp.float32),
        mesh=_mesh(),
        compiler_params=pltpu.CompilerParams(use_tc_tiling_on_sc=False),
        scratch_types=[
            pltpu.VMEM((NCHUNK, CHUNK), jnp.int32),
            pltpu.VMEM((CHUNK, DEGW), jnp.float32),
            pltpu.VMEM((ROWS_PT, DEGW), jnp.float32),
            pltpu.VMEM_SHARED((NP, DEGW), jnp.float32),
            pltpu.SemaphoreType.DMA,
        ],
    )(dst3d)


def _make_prop_body(dd):
    ng = NCHUNK // dd

    def _prop_body(hs_hbm, src3d_hbm, dst3d_hbm, part_hbm,
                   srcb, dstb, rows, acc_sp, g0, g1, s0, s1):
        c = lax.axis_index("c")
        s = lax.axis_index("s")
        wid = c * NS + s

        pltpu.sync_copy(hs_hbm.at[pl.ds(s * ROWS_PT, ROWS_PT)],
                        acc_sp.at[pl.ds(s * ROWS_PT, ROWS_PT)])
        pltpu.sync_copy(src3d_hbm.at[wid], srcb)
        pltpu.sync_copy(dst3d_hbm.at[wid], dstb)
        plsc.subcore_barrier()

        for d in range(dd):
            pltpu.async_copy(hs_hbm.at[srcb.at[d]], rows.at[d], g0)
        for d in range(dd):
            pltpu.async_copy(hs_hbm.at[srcb.at[dd + d]], rows.at[dd + d], g1)

        def _phase(j, j_next, off, gsem, ssem):
            for d in range(dd):
                pltpu.make_async_copy(hs_hbm.at[srcb.at[j + d]],
                                      rows.at[off + d], gsem).wait()
            for d in range(dd):
                pltpu.async_copy(rows.at[off + d],
                                 acc_sp.at[dstb.at[j + d]], ssem, add=True)
            for d in range(dd):
                pltpu.make_async_copy(rows.at[off + d],
                                      acc_sp.at[dstb.at[j + d]],
                                      ssem).wait()
            for d in range(dd):
                pltpu.async_copy(hs_hbm.at[srcb.at[j_next + d]],
                                 rows.at[off + d], gsem)

        def body(i, _):
            jA = (2 * i) * dd
            jB = jA + dd
            jB2 = jnp.where(jA + 3 * dd >= NCHUNK, 0, jA + 3 * dd)
            _phase(jA, jA + 2 * dd, 0, g0, s0)
            _phase(jB, jB2, dd, g1, s1)
            return 0
        lax.fori_loop(0, (ng - 1) // 2, body, 0)

        last = (ng - 1) * dd
        for d in range(dd):
            pltpu.make_async_copy(hs_hbm.at[srcb.at[last + d]],
                                  rows.at[d], g0).wait()
        for d in range(dd):
            pltpu.async_copy(rows.at[d], acc_sp.at[dstb.at[last + d]],
                             s0, add=True)
        for d in range(dd):
            pltpu.make_async_copy(rows.at[d], acc_sp.at[dstb.at[last + d]],
                                  s0).wait()
        for d in range(dd):
            pltpu.make_async_copy(hs_hbm.at[srcb.at[d]],
                                  rows.at[dd + d], g1).wait()

        plsc.subcore_barrier()
        pltpu.sync_copy(acc_sp.at[pl.ds(s * ROWS_PT, ROWS_PT)],
                        part_hbm.at[c, pl.ds(s * ROWS_PT, ROWS_PT)])

    return _prop_body


def _prop(hs, src3d, dst3d, w):
    dd = D if w == 64 else 1
    return pl.kernel(
        _make_prop_body(dd),
        out_type=jax.ShapeDtypeStruct((NC, NP, w), jnp.float32),
        mesh=_mesh(),
        compiler_params=pltpu.CompilerParams(use_tc_tiling_on_sc=False),
        scratch_types=[
            pltpu.VMEM((NCHUNK, CHUNK), jnp.int32),
            pltpu.VMEM((NCHUNK, CHUNK), jnp.int32),
            pltpu.VMEM((2 * dd, CHUNK, w), jnp.float32),
            pltpu.VMEM_SHARED((NP, w), jnp.float32),
            pltpu.SemaphoreType.DMA,
            pltpu.SemaphoreType.DMA,
            pltpu.SemaphoreType.DMA,
            pltpu.SemaphoreType.DMA,
        ],
    )(hs, src3d, dst3d)


def _dinv_from(degp):
    deg = degp[0] + degp[1]
    return lax.rsqrt(deg[:, :1] + 1.0)


def _mm_scale_body(x_ref, w_ref, degp_ref, hs_ref):
    dinv = _dinv_from(degp_ref[...])
    xw = jnp.dot(x_ref[...], w_ref[...], preferred_element_type=jnp.float32)
    hs_ref[pl.ds(0, N), :] = xw * dinv[:N]
    hs_ref[pl.ds(N, PAD), :] = jnp.zeros((PAD, 64), jnp.float32)


def _combine_mm_body(a_ref, hs_ref, degp_ref, b1_ref, w2_ref, out_ref):
    dinv = _dinv_from(degp_ref[...])
    a = a_ref[...]
    hs = hs_ref[...]
    p = (a[0] + a[1] - hs) * dinv + b1_ref[...]
    h1 = jnp.maximum(p, 0.0)
    hh = jnp.dot(h1, w2_ref[...], preferred_element_type=jnp.float32)
    out_ref[...] = hh * dinv
    out_ref[pl.ds(N, PAD), :] = jnp.zeros((PAD, 128), jnp.float32)


def _leaky(v, alpha):
    return jnp.where(v > 0, v, alpha * v)


def _head_body(a_ref, hs_ref, degp_ref, b2_ref, batch_ref,
               m0_ref, mb0_ref, m1_ref, mb1_ref, m2_ref, mb2_ref,
               m3_ref, mb3_ref, out_ref):
    dinv = _dinv_from(degp_ref[...])
    a = a_ref[...]
    p2 = (a[0] + a[1] - hs_ref[...]) * dinv + b2_ref[...]
    h2 = jnp.maximum(p2, 0.0)
    gids = lax.broadcasted_iota(jnp.int32, (NUM_GRAPHS, NP), 0)
    onehot = (batch_ref[...] == gids).astype(jnp.float32)
    sums = jnp.dot(onehot, h2, preferred_element_type=jnp.float32,
                   precision=lax.Precision.HIGHEST)
    cnt = jnp.sum(onehot, axis=1, keepdims=True)
    g = sums / jnp.maximum(cnt, 1.0)
    g = _leaky(jnp.dot(g, m0_ref[...], preferred_element_type=jnp.float32)
               + mb0_ref[...], 0.2)
    g = _leaky(jnp.dot(g, m1_ref[...], preferred_element_type=jnp.float32)
               + mb1_ref[...], 0.1)
    g = _leaky(jnp.dot(g, m2_ref[...], preferred_element_type=jnp.float32)
               + mb2_ref[...], 0.1)
    g = jnp.dot(g, m3_ref[...], preferred_element_type=jnp.float32)
    out_ref[...] = jnp.maximum(g + mb3_ref[...], 0.0)


def _tc_call(body, out_shape, *args):
    return pl.pallas_call(
        body,
        out_shape=jax.ShapeDtypeStruct(out_shape, jnp.float32),
    )(*args)


def kernel(x, edge_index, batch, W1, b1, W2, b2,
           M0, mb0, M1, mb1, M2, mb2, M3, mb3):
    src3d = edge_index[0].reshape(NW, NCHUNK, CHUNK)
    dst3d = edge_index[1].reshape(NW, NCHUNK, CHUNK)
    batch_pad = jnp.concatenate(
        [batch, jnp.full((PAD,), -1, jnp.int32)]).reshape(1, NP)

    degp = _deg(dst3d)
    hs1 = _tc_call(_mm_scale_body, (NP, 64),
                   x, W1, degp)
    part1 = _prop(hs1, src3d, dst3d, 64)
    hs2 = _tc_call(_combine_mm_body, (NP, 128),
                   part1, hs1, degp, b1.reshape(1, 64),
                   W2)
    part2 = _prop(hs2, src3d, dst3d, 128)
    out = _tc_call(_head_body, (NUM_GRAPHS, 1),
                   part2, hs2, degp, b2.reshape(1, 128), batch_pad,
                   M0, mb0.reshape(1, 64), M1, mb1.reshape(1, 64),
                   M2, mb2.reshape(1, 64), M3, mb3.reshape(1, 1))
    return out
